# baseline (speedup 1.0000x reference)
import functools

import jax
import jax.numpy as jnp
from jax import lax
from jax.experimental import pallas as pl
from jax.experimental.pallas import tpu as pltpu
from jax.experimental.pallas import tpu_sc as plsc

N_ITEMS = 30490
N_DAYS = 1969

_CB = 128
_GRID = 239
_ITEMS_PAD = _CB * _GRID
_DPAD = 2048
_TBLK = _DPAD * _CB
_TSIZE = _GRID * _TBLK

_NW = 32
_LANES = 16
_CHUNK = 3200
_MAGIC = 10486
_MSHIFT = 21


def _tc_body(pt_ref, mean_ref, tbl_ref):
    x = pt_ref[...]
    mean_ref[...] = jnp.mean(x, axis=0)
    xp = jnp.concatenate(
        [x, jnp.zeros((_DPAD - N_DAYS, _CB), jnp.float32)], axis=0)
    tbl_ref[...] = xp.reshape(_TBLK)


def _mean_and_flat(prices_t):
    return pl.pallas_call(
        _tc_body,
        grid=(_GRID,),
        in_specs=[pl.BlockSpec((N_DAYS, _CB), lambda i: (0, i))],
        out_specs=[
            pl.BlockSpec((_CB,), lambda i: (i,)),
            pl.BlockSpec((_TBLK,), lambda i: (i,)),
        ],
        out_shape=[
            jax.ShapeDtypeStruct((_ITEMS_PAD,), jnp.float32),
            jax.ShapeDtypeStruct((_TSIZE,), jnp.float32),
        ],
    )(prices_t)


def _sc_body(per_w, tbl_hbm, days2_hbm, items2_hbm, mean_hbm,
             op2_hbm, om2_hbm, or2_hbm, mean_v,
             days_a, items_a, price_a, idx_a, pg_a, mg_a, rg_a, sem_a,
             days_b, items_b, price_b, idx_b, pg_b, mg_b, rg_b, sem_b):
    rows, cols = days2_hbm.shape
    crows = _CHUNK // cols
    nch = rows // crows
    days_hbm = days2_hbm.reshape(nch, crows, cols)
    items_hbm = items2_hbm.reshape(nch, crows, cols)
    op_hbm = op2_hbm.reshape(nch, crows, cols)
    om_hbm = om2_hbm.reshape(nch, crows, cols)
    or_hbm = or2_hbm.reshape(nch, crows, cols)
    wid = lax.axis_index("s") * 2 + lax.axis_index("c")
    n_chunks = per_w // _CHUNK
    base = wid * n_chunks
    pltpu.sync_copy(mean_hbm, mean_v)
    lanes = lax.iota(jnp.int32, _LANES)
    bufs = (
        (days_a, items_a, price_a, idx_a, pg_a, mg_a, rg_a, sem_a),
        (days_b, items_b, price_b, idx_b, pg_b, mg_b, rg_b, sem_b),
    )

    def rc(i):
        j = i * _LANES + lanes
        r = (j * _MAGIC) >> _MSHIFT
        return r, j - r * cols

    def stage(c, buf):
        days_v, items_v, price_v, idx_v, _, mg_v, _, sem = bufs[buf]
        off = base + c
        pltpu.sync_copy(days_hbm.at[off], days_v)
        pltpu.sync_copy(items_hbm.at[off], items_v)

        def idx_loop(i, carry):
            r, cc = rc(i)
            it = plsc.load_gather(items_v, [r, cc])
            dy = plsc.load_gather(days_v, [r, cc])
            idx_v[pl.ds(i * _LANES, _LANES)] = (
                ((it >> 7) << 18) + ((dy >> 3) << 10)
                + ((dy & 7) << 7) + (it & 127)
            )
            plsc.store_scatter(mg_v, [r, cc], plsc.load_gather(mean_v, [it]))
            return carry

        lax.fori_loop(0, _CHUNK // _LANES, idx_loop, 0)
        pltpu.async_copy(tbl_hbm.at[idx_v], price_v, sem)

    def drain(c, buf):
        _, _, price_v, idx_v, pg_v, mg_v, rg_v, sem = bufs[buf]
        off = base + c
        pltpu.make_async_copy(tbl_hbm.at[idx_v], price_v, sem).wait()

        def div_loop(i, carry):
            r, cc = rc(i)
            p = price_v[pl.ds(i * _LANES, _LANES)]
            m = plsc.load_gather(mg_v, [r, cc])
            plsc.store_scatter(pg_v, [r, cc], p)
            plsc.store_scatter(rg_v, [r, cc], p / m)
            return carry

        lax.fori_loop(0, _CHUNK // _LANES, div_loop, 0)
        pltpu.sync_copy(pg_v, op_hbm.at[off])
        pltpu.sync_copy(mg_v, om_hbm.at[off])
        pltpu.sync_copy(rg_v, or_hbm.at[off])

    stage(0, 0)
    stage(1, 1)

    def step(g, carry):
        c = g * 2
        drain(c, 0)
        stage(c + 2, 0)
        drain(c + 1, 1)
        stage(c + 3, 1)
        return carry

    lax.fori_loop(0, n_chunks // 2 - 1, step, 0)
    drain(n_chunks - 2, 0)
    drain(n_chunks - 1, 1)


@functools.partial(jax.jit, static_argnames=("b", "l"))
def _sc_gather(tbl_flat, days2, items2, mean_pad, *, b, l):
    per_w = b * l // _NW
    assert per_w % _CHUNK == 0
    mesh = plsc.VectorSubcoreMesh(core_axis_name="c", subcore_axis_name="s")
    out = jax.ShapeDtypeStruct((b, l), jnp.float32)
    k = pl.kernel(
        functools.partial(_sc_body, per_w),
        out_type=(out, out, out),
        mesh=mesh,
        compiler_params=pltpu.CompilerParams(needs_layout_passes=False),
        scratch_types=[
            pltpu.VMEM((_ITEMS_PAD,), jnp.float32),
        ] + 2 * [
            pltpu.VMEM((_CHUNK // l, l), jnp.int32),
            pltpu.VMEM((_CHUNK // l, l), jnp.int32),
            pltpu.VMEM((_CHUNK,), jnp.float32),
            pltpu.VMEM((_CHUNK,), jnp.int32),
            pltpu.VMEM((_CHUNK // l, l), jnp.float32),
            pltpu.VMEM((_CHUNK // l, l), jnp.float32),
            pltpu.VMEM((_CHUNK // l, l), jnp.float32),
            pltpu.SemaphoreType.DMA,
        ],
    )
    return k(tbl_flat, days2, items2, mean_pad)


def kernel(prices, days_index, items_index):
    b, l = days_index.shape
    mean_pad, tbl_flat = _mean_and_flat(prices.T)
    return _sc_gather(
        tbl_flat,
        days_index.astype(jnp.int32),
        items_index.astype(jnp.int32),
        mean_pad,
        b=b, l=l,
    )


# speedup vs baseline: 1.3418x; 1.3418x over previous
import functools

import jax
import jax.numpy as jnp
from jax import lax
from jax.experimental import pallas as pl
from jax.experimental.pallas import tpu as pltpu
from jax.experimental.pallas import tpu_sc as plsc

N_ITEMS = 30490
N_DAYS = 1969

_CB = 128
_GRID = 239
_ITEMS_PAD = _CB * _GRID
_DPAD = 2048
_TBLK = _DPAD * _CB
_TSIZE = _GRID * _TBLK

_NW = 32
_LANES = 16
_CR = 8
_CW = 512
_CHUNK = _CR * _CW


def _tc_body(pt_ref, mean_ref, tbl_ref):
    x = pt_ref[...]
    mean_ref[...] = jnp.mean(x, axis=0)
    xp = jnp.concatenate(
        [x, jnp.zeros((_DPAD - N_DAYS, _CB), jnp.float32)], axis=0)
    tbl_ref[...] = xp.reshape(_TBLK)


def _mean_and_flat(prices_t):
    return pl.pallas_call(
        _tc_body,
        grid=(_GRID,),
        in_specs=[pl.BlockSpec((N_DAYS, _CB), lambda i: (0, i))],
        out_specs=[
            pl.BlockSpec((_CB,), lambda i: (i,)),
            pl.BlockSpec((_TBLK,), lambda i: (i,)),
        ],
        out_shape=[
            jax.ShapeDtypeStruct((_ITEMS_PAD,), jnp.float32),
            jax.ShapeDtypeStruct((_TSIZE,), jnp.float32),
        ],
    )(prices_t)


def _sc_body(n_chunks, tbl_hbm, days_hbm, items_hbm, mean_hbm,
             op_hbm, om_hbm, or_hbm, mean_v,
             days_a, items_a, price_a, idx_a, pg_a, mg_a, rg_a, sem_a,
             days_b, items_b, price_b, idx_b, pg_b, mg_b, rg_b, sem_b):
    wid = lax.axis_index("s") * 2 + lax.axis_index("c")
    b0 = wid * _CW
    pltpu.sync_copy(mean_hbm, mean_v)
    bufs = (
        (days_a, items_a, price_a, idx_a, pg_a, mg_a, rg_a, sem_a),
        (days_b, items_b, price_b, idx_b, pg_b, mg_b, rg_b, sem_b),
    )
    gpr = _CW // _LANES

    def stage(c, buf):
        days_v, items_v, price_v, idx_v, _, mg_v, _, sem = bufs[buf]
        sl = (pl.ds(c * _CR, _CR), pl.ds(b0, _CW))
        pltpu.sync_copy(days_hbm.at[sl], days_v)
        pltpu.sync_copy(items_hbm.at[sl], items_v)

        def idx_loop(i, carry):
            u = i // gpr
            s = pl.ds((i % gpr) * _LANES, _LANES)
            it = items_v[u, s]
            dy = days_v[u, s]
            idx_v[pl.ds(i * _LANES, _LANES)] = (
                ((it >> 7) << 18) + ((dy >> 3) << 10)
                + ((dy & 7) << 7) + (it & 127)
            )
            mg_v[u, s] = plsc.load_gather(mean_v, [it])
            return carry

        lax.fori_loop(0, _CHUNK // _LANES, idx_loop, 0)
        pltpu.async_copy(tbl_hbm.at[idx_v], price_v, sem)

    def drain(c, buf):
        _, _, price_v, idx_v, pg_v, mg_v, rg_v, sem = bufs[buf]
        sl = (pl.ds(c * _CR, _CR), pl.ds(b0, _CW))
        pltpu.make_async_copy(tbl_hbm.at[idx_v], price_v, sem).wait()

        def div_loop(i, carry):
            u = i // gpr
            s = pl.ds((i % gpr) * _LANES, _LANES)
            p = price_v[pl.ds(i * _LANES, _LANES)]
            pg_v[u, s] = p
            rg_v[u, s] = p / mg_v[u, s]
            return carry

        lax.fori_loop(0, _CHUNK // _LANES, div_loop, 0)
        pltpu.sync_copy(pg_v, op_hbm.at[sl])
        pltpu.sync_copy(mg_v, om_hbm.at[sl])
        pltpu.sync_copy(rg_v, or_hbm.at[sl])

    stage(0, 0)
    stage(1, 1)

    def step(g, carry):
        c = g * 2
        drain(c, 0)
        stage(c + 2, 0)
        drain(c + 1, 1)
        stage(c + 3, 1)
        return carry

    pairs = (n_chunks - 3) // 2 if n_chunks % 2 else n_chunks // 2 - 1
    lax.fori_loop(0, pairs, step, 0)
    if n_chunks % 2:
        drain(n_chunks - 3, 0)
        stage(n_chunks - 1, 0)
        drain(n_chunks - 2, 1)
        drain(n_chunks - 1, 0)
    else:
        drain(n_chunks - 2, 0)
        drain(n_chunks - 1, 1)


@functools.partial(jax.jit, static_argnames=("l", "b"))
def _sc_gather(tbl_flat, days_t, items_t, mean_pad, *, l, b):
    assert l % _CR == 0 and b == _CW * _NW
    n_chunks = l // _CR
    mesh = plsc.VectorSubcoreMesh(core_axis_name="c", subcore_axis_name="s")
    out = jax.ShapeDtypeStruct((l, b), jnp.float32)
    k = pl.kernel(
        functools.partial(_sc_body, n_chunks),
        out_type=(out, out, out),
        mesh=mesh,
        compiler_params=pltpu.CompilerParams(needs_layout_passes=False),
        scratch_types=[
            pltpu.VMEM((_ITEMS_PAD,), jnp.float32),
        ] + 2 * [
            pltpu.VMEM((_CR, _CW), jnp.int32),
            pltpu.VMEM((_CR, _CW), jnp.int32),
            pltpu.VMEM((_CHUNK,), jnp.float32),
            pltpu.VMEM((_CHUNK,), jnp.int32),
            pltpu.VMEM((_CR, _CW), jnp.float32),
            pltpu.VMEM((_CR, _CW), jnp.float32),
            pltpu.VMEM((_CR, _CW), jnp.float32),
            pltpu.SemaphoreType.DMA,
        ],
    )
    return k(tbl_flat, days_t, items_t, mean_pad)


def kernel(prices, days_index, items_index):
    b, l = days_index.shape
    mean_pad, tbl_flat = _mean_and_flat(prices.T)
    gp, gm, gr = _sc_gather(
        tbl_flat,
        days_index.T.astype(jnp.int32),
        items_index.T.astype(jnp.int32),
        mean_pad,
        l=l, b=b,
    )
    return gp.T, gm.T, gr.T


# speedup vs baseline: 1.4143x; 1.0541x over previous
import functools

import jax
import jax.numpy as jnp
from jax import lax
from jax.experimental import pallas as pl
from jax.experimental.pallas import tpu as pltpu
from jax.experimental.pallas import tpu_sc as plsc

N_ITEMS = 30490
N_DAYS = 1969

_CB = 128
_GRID = 239
_ITEMS_PAD = _CB * _GRID
_DPAD = 2048
_TBLK = _DPAD * _CB
_TSIZE = _GRID * _TBLK

_NW = 32
_LANES = 16
_CR = 8
_CW = 512
_CHUNK = _CR * _CW


def _tc_body(pt_ref, mean_ref, tbl_ref):
    x = pt_ref[...]
    mean_ref[...] = jnp.mean(x, axis=0)
    xp = jnp.concatenate(
        [x, jnp.zeros((_DPAD - N_DAYS, _CB), jnp.float32)], axis=0)
    tbl_ref[...] = xp.reshape(_TBLK)


def _mean_and_flat(prices_t):
    return pl.pallas_call(
        _tc_body,
        grid=(_GRID,),
        in_specs=[pl.BlockSpec((N_DAYS, _CB), lambda i: (0, i))],
        out_specs=[
            pl.BlockSpec((_CB,), lambda i: (i,)),
            pl.BlockSpec((_TBLK,), lambda i: (i,)),
        ],
        out_shape=[
            jax.ShapeDtypeStruct((_ITEMS_PAD,), jnp.float32),
            jax.ShapeDtypeStruct((_TSIZE,), jnp.float32),
        ],
    )(prices_t)


def _sc_body(n_chunks, tbl_hbm, days_hbm, items_hbm, mean_hbm,
             op_hbm, om_hbm, or_hbm, mean_v,
             days_a, items_a, price_a, idx_a, pg_a, mg_a, rg_a,
             gsem_a, isem_a, msem_a, osem_a,
             days_b, items_b, price_b, idx_b, pg_b, mg_b, rg_b,
             gsem_b, isem_b, msem_b, osem_b):
    wid = lax.axis_index("s") * 2 + lax.axis_index("c")
    b0 = wid * _CW
    pltpu.sync_copy(mean_hbm, mean_v)
    bufs = (
        (days_a, items_a, price_a, idx_a, pg_a, mg_a, rg_a,
         gsem_a, isem_a, msem_a, osem_a),
        (days_b, items_b, price_b, idx_b, pg_b, mg_b, rg_b,
         gsem_b, isem_b, msem_b, osem_b),
    )
    gpr = _CW // _LANES

    def sl(c):
        return (pl.ds(c * _CR, _CR), pl.ds(b0, _CW))

    def prefetch(c):
        days_v, items_v, *_, isem, _, _ = bufs[c % 2]
        pltpu.async_copy(days_hbm.at[sl(c)], days_v, isem)
        pltpu.async_copy(items_hbm.at[sl(c)], items_v, isem)

    def stage(c):
        days_v, items_v, price_v, idx_v, _, mg_v, _, gsem, isem, msem, _ = (
            bufs[c % 2])
        pltpu.make_async_copy(days_hbm.at[sl(c)], days_v, isem).wait()
        pltpu.make_async_copy(items_hbm.at[sl(c)], items_v, isem).wait()
        if c >= 2:
            pltpu.make_async_copy(mg_v, om_hbm.at[sl(c)], msem).wait()

        def idx_loop(i, carry):
            u = i // gpr
            s = pl.ds((i % gpr) * _LANES, _LANES)
            it = items_v[u, s]
            dy = days_v[u, s]
            idx_v[pl.ds(i * _LANES, _LANES)] = (
                ((it >> 7) << 18) + ((dy >> 3) << 10)
                + ((dy & 7) << 7) + (it & 127)
            )
            mg_v[u, s] = plsc.load_gather(mean_v, [it])
            return carry

        lax.fori_loop(0, _CHUNK // _LANES, idx_loop, 0)
        pltpu.async_copy(tbl_hbm.at[idx_v], price_v, gsem)
        pltpu.async_copy(mg_v, om_hbm.at[sl(c)], msem)

    def drain(c):
        _, _, price_v, idx_v, pg_v, mg_v, rg_v, gsem, _, _, osem = bufs[c % 2]
        pltpu.make_async_copy(tbl_hbm.at[idx_v], price_v, gsem).wait()
        if c >= 2:
            pltpu.make_async_copy(pg_v, op_hbm.at[sl(c)], osem).wait()
            pltpu.make_async_copy(rg_v, or_hbm.at[sl(c)], osem).wait()

        def div_loop(i, carry):
            u = i // gpr
            s = pl.ds((i % gpr) * _LANES, _LANES)
            p = price_v[pl.ds(i * _LANES, _LANES)]
            pg_v[u, s] = p
            rg_v[u, s] = p / mg_v[u, s]
            return carry

        lax.fori_loop(0, _CHUNK // _LANES, div_loop, 0)
        pltpu.async_copy(pg_v, op_hbm.at[sl(c)], osem)
        pltpu.async_copy(rg_v, or_hbm.at[sl(c)], osem)

    prefetch(0)
    prefetch(1)
    stage(0)
    stage(1)
    for c in range(n_chunks):
        if c + 2 < n_chunks:
            prefetch(c + 2)
        drain(c)
        if c + 2 < n_chunks:
            stage(c + 2)
    for c in (n_chunks - 2, n_chunks - 1):
        _, _, _, _, pg_v, mg_v, rg_v, _, _, msem, osem = bufs[c % 2]
        pltpu.make_async_copy(mg_v, om_hbm.at[sl(c)], msem).wait()
        pltpu.make_async_copy(pg_v, op_hbm.at[sl(c)], osem).wait()
        pltpu.make_async_copy(rg_v, or_hbm.at[sl(c)], osem).wait()


@functools.partial(jax.jit, static_argnames=("l", "b"))
def _sc_gather(tbl_flat, days_t, items_t, mean_pad, *, l, b):
    assert l % _CR == 0 and b == _CW * _NW
    n_chunks = l // _CR
    mesh = plsc.VectorSubcoreMesh(core_axis_name="c", subcore_axis_name="s")
    out = jax.ShapeDtypeStruct((l, b), jnp.float32)
    k = pl.kernel(
        functools.partial(_sc_body, n_chunks),
        out_type=(out, out, out),
        mesh=mesh,
        compiler_params=pltpu.CompilerParams(needs_layout_passes=False),
        scratch_types=[
            pltpu.VMEM((_ITEMS_PAD,), jnp.float32),
        ] + 2 * [
            pltpu.VMEM((_CR, _CW), jnp.int32),
            pltpu.VMEM((_CR, _CW), jnp.int32),
            pltpu.VMEM((_CHUNK,), jnp.float32),
            pltpu.VMEM((_CHUNK,), jnp.int32),
            pltpu.VMEM((_CR, _CW), jnp.float32),
            pltpu.VMEM((_CR, _CW), jnp.float32),
            pltpu.VMEM((_CR, _CW), jnp.float32),
            pltpu.SemaphoreType.DMA,
            pltpu.SemaphoreType.DMA,
            pltpu.SemaphoreType.DMA,
            pltpu.SemaphoreType.DMA,
        ],
    )
    return k(tbl_flat, days_t, items_t, mean_pad)


def kernel(prices, days_index, items_index):
    b, l = days_index.shape
    mean_pad, tbl_flat = _mean_and_flat(prices.T)
    gp, gm, gr = _sc_gather(
        tbl_flat,
        days_index.T.astype(jnp.int32),
        items_index.T.astype(jnp.int32),
        mean_pad,
        l=l, b=b,
    )
    return gp.T, gm.T, gr.T


# speedup vs baseline: 1.5829x; 1.1192x over previous
import functools

import jax
import jax.numpy as jnp
from jax import lax
from jax.experimental import pallas as pl
from jax.experimental.pallas import tpu as pltpu
from jax.experimental.pallas import tpu_sc as plsc

N_ITEMS = 30490
N_DAYS = 1969

_CB = 256
_GRID = 120
_ITEMS_PAD = _CB * _GRID
_DPAD = 2048
_TBLK = _DPAD * _CB
_TSIZE = _GRID * _TBLK

_NW = 32
_LANES = 16
_CR = 8
_CW = 512
_CHUNK = _CR * _CW


def _tc_body(pt_ref, mean_ref, tbl_ref):
    x = pt_ref[...]
    mean_ref[...] = jnp.mean(x, axis=0)
    xp = jnp.concatenate(
        [x, jnp.zeros((_DPAD - N_DAYS, _CB), jnp.float32)], axis=0)
    y = xp.reshape(_DPAD // 8, 8, _CB // 128, 128).transpose(0, 2, 1, 3)
    tbl_ref[...] = y.reshape(_TBLK)


def _mean_and_flat(prices_t):
    return pl.pallas_call(
        _tc_body,
        grid=(_GRID,),
        in_specs=[pl.BlockSpec((N_DAYS, _CB), lambda i: (0, i))],
        out_specs=[
            pl.BlockSpec((_CB,), lambda i: (i,)),
            pl.BlockSpec((_TBLK,), lambda i: (i,)),
        ],
        out_shape=[
            jax.ShapeDtypeStruct((_ITEMS_PAD,), jnp.float32),
            jax.ShapeDtypeStruct((_TSIZE,), jnp.float32),
        ],
    )(prices_t)


def _sc_body(n_chunks, tbl_hbm, days_hbm, items_hbm, mean_hbm,
             op_hbm, om_hbm, or_hbm, mean_v,
             days_a, items_a, price_a, idx_a, pg_a, mg_a, rg_a,
             gsem_a, isem_a, msem_a, osem_a,
             days_b, items_b, price_b, idx_b, pg_b, mg_b, rg_b,
             gsem_b, isem_b, msem_b, osem_b):
    wid = lax.axis_index("s") * 2 + lax.axis_index("c")
    b0 = wid * _CW
    pltpu.sync_copy(mean_hbm, mean_v)
    bufs = (
        (days_a, items_a, price_a, idx_a, pg_a, mg_a, rg_a,
         gsem_a, isem_a, msem_a, osem_a),
        (days_b, items_b, price_b, idx_b, pg_b, mg_b, rg_b,
         gsem_b, isem_b, msem_b, osem_b),
    )
    gpr = _CW // _LANES

    def sl(c):
        return (pl.ds(c * _CR, _CR), pl.ds(b0, _CW))

    def prefetch(c):
        days_v, items_v, *_, isem, _, _ = bufs[c % 2]
        pltpu.async_copy(days_hbm.at[sl(c)], days_v, isem)
        pltpu.async_copy(items_hbm.at[sl(c)], items_v, isem)

    def stage(c):
        days_v, items_v, price_v, idx_v, _, mg_v, _, gsem, isem, msem, _ = (
            bufs[c % 2])
        pltpu.make_async_copy(days_hbm.at[sl(c)], days_v, isem).wait()
        pltpu.make_async_copy(items_hbm.at[sl(c)], items_v, isem).wait()
        if c >= 2:
            pltpu.make_async_copy(mg_v, om_hbm.at[sl(c)], msem).wait()

        def idx_loop(i, carry):
            u = i // gpr
            s = pl.ds((i % gpr) * _LANES, _LANES)
            it = items_v[u, s]
            dy = days_v[u, s]
            idx_v[pl.ds(i * _LANES, _LANES)] = (
                ((it >> 8) << 19) + ((dy >> 3) << 11)
                + (((it >> 7) & 1) << 10) + ((dy & 7) << 7) + (it & 127)
            )
            mg_v[u, s] = plsc.load_gather(mean_v, [it])
            return carry

        lax.fori_loop(0, _CHUNK // _LANES, idx_loop, 0)
        pltpu.async_copy(tbl_hbm.at[idx_v], price_v, gsem)
        pltpu.async_copy(mg_v, om_hbm.at[sl(c)], msem)

    def drain(c):
        _, _, price_v, idx_v, pg_v, mg_v, rg_v, gsem, _, _, osem = bufs[c % 2]
        pltpu.make_async_copy(tbl_hbm.at[idx_v], price_v, gsem).wait()
        if c >= 2:
            pltpu.make_async_copy(pg_v, op_hbm.at[sl(c)], osem).wait()
            pltpu.make_async_copy(rg_v, or_hbm.at[sl(c)], osem).wait()

        def div_loop(i, carry):
            u = i // gpr
            s = pl.ds((i % gpr) * _LANES, _LANES)
            p = price_v[pl.ds(i * _LANES, _LANES)]
            pg_v[u, s] = p
            rg_v[u, s] = p / mg_v[u, s]
            return carry

        lax.fori_loop(0, _CHUNK // _LANES, div_loop, 0)
        pltpu.async_copy(pg_v, op_hbm.at[sl(c)], osem)
        pltpu.async_copy(rg_v, or_hbm.at[sl(c)], osem)

    prefetch(0)
    prefetch(1)
    stage(0)
    stage(1)
    for c in range(n_chunks):
        if c + 2 < n_chunks:
            prefetch(c + 2)
        drain(c)
        if c + 2 < n_chunks:
            stage(c + 2)
    for c in (n_chunks - 2, n_chunks - 1):
        _, _, _, _, pg_v, mg_v, rg_v, _, _, msem, osem = bufs[c % 2]
        pltpu.make_async_copy(mg_v, om_hbm.at[sl(c)], msem).wait()
        pltpu.make_async_copy(pg_v, op_hbm.at[sl(c)], osem).wait()
        pltpu.make_async_copy(rg_v, or_hbm.at[sl(c)], osem).wait()


@functools.partial(jax.jit, static_argnames=("l", "b"))
def _sc_gather(tbl_flat, days_t, items_t, mean_pad, *, l, b):
    assert l % _CR == 0 and b == _CW * _NW
    n_chunks = l // _CR
    mesh = plsc.VectorSubcoreMesh(core_axis_name="c", subcore_axis_name="s")
    out = jax.ShapeDtypeStruct((l, b), jnp.float32)
    k = pl.kernel(
        functools.partial(_sc_body, n_chunks),
        out_type=(out, out, out),
        mesh=mesh,
        compiler_params=pltpu.CompilerParams(needs_layout_passes=False),
        scratch_types=[
            pltpu.VMEM((_ITEMS_PAD,), jnp.float32),
        ] + 2 * [
            pltpu.VMEM((_CR, _CW), jnp.int32),
            pltpu.VMEM((_CR, _CW), jnp.int32),
            pltpu.VMEM((_CHUNK,), jnp.float32),
            pltpu.VMEM((_CHUNK,), jnp.int32),
            pltpu.VMEM((_CR, _CW), jnp.float32),
            pltpu.VMEM((_CR, _CW), jnp.float32),
            pltpu.VMEM((_CR, _CW), jnp.float32),
            pltpu.SemaphoreType.DMA,
            pltpu.SemaphoreType.DMA,
            pltpu.SemaphoreType.DMA,
            pltpu.SemaphoreType.DMA,
        ],
    )
    return k(tbl_flat, days_t, items_t, mean_pad)


def kernel(prices, days_index, items_index):
    b, l = days_index.shape
    mean_pad, tbl_flat = _mean_and_flat(prices.T)
    gp, gm, gr = _sc_gather(
        tbl_flat,
        days_index.T.astype(jnp.int32),
        items_index.T.astype(jnp.int32),
        mean_pad,
        l=l, b=b,
    )
    return gp.T, gm.T, gr.T


# speedup vs baseline: 1.8215x; 1.1507x over previous
import functools

import jax
import jax.numpy as jnp
from jax import lax
from jax.experimental import pallas as pl
from jax.experimental.pallas import tpu as pltpu
from jax.experimental.pallas import tpu_sc as plsc

N_ITEMS = 30490
N_DAYS = 1969

_CB = 512
_GRID = 60
_ITEMS_PAD = _CB * _GRID
_DPAD = 2048
_TBLK = _DPAD * _CB
_TSIZE = _GRID * _TBLK

_NW = 32
_LANES = 16
_CR = 8
_CW = 512
_CHUNK = _CR * _CW


def _tc_body(pt_ref, mean_ref, tbl_ref):
    x = pt_ref[...]
    mean_ref[...] = jnp.mean(x, axis=0)
    xp = jnp.concatenate(
        [x, jnp.zeros((_DPAD - N_DAYS, _CB), jnp.float32)], axis=0)
    y = xp.reshape(_DPAD // 8, 8, _CB // 128, 128).transpose(0, 2, 1, 3)
    tbl_ref[...] = y.reshape(_TBLK)


def _mean_and_flat(prices_t):
    return pl.pallas_call(
        _tc_body,
        grid=(_GRID,),
        in_specs=[pl.BlockSpec((N_DAYS, _CB), lambda i: (0, i))],
        out_specs=[
            pl.BlockSpec((_CB,), lambda i: (i,)),
            pl.BlockSpec((_TBLK,), lambda i: (i,)),
        ],
        out_shape=[
            jax.ShapeDtypeStruct((_ITEMS_PAD,), jnp.float32),
            jax.ShapeDtypeStruct((_TSIZE,), jnp.float32),
        ],
    )(prices_t)


def _sc_body(n_chunks, tbl_hbm, days_hbm, items_hbm, mean_hbm,
             op_hbm, om_hbm, or_hbm, mean_v,
             days_a, items_a, price_a, idx_a, pg_a, mg_a, rg_a,
             gsem_a, isem_a, msem_a, osem_a,
             days_b, items_b, price_b, idx_b, pg_b, mg_b, rg_b,
             gsem_b, isem_b, msem_b, osem_b):
    wid = lax.axis_index("s") * 2 + lax.axis_index("c")
    b0 = wid * _CW
    pltpu.sync_copy(mean_hbm, mean_v)
    bufs = (
        (days_a, items_a, price_a, idx_a, pg_a, mg_a, rg_a,
         gsem_a, isem_a, msem_a, osem_a),
        (days_b, items_b, price_b, idx_b, pg_b, mg_b, rg_b,
         gsem_b, isem_b, msem_b, osem_b),
    )
    gpr = _CW // _LANES

    def sl(c):
        return (pl.ds(c * _CR, _CR), pl.ds(b0, _CW))

    def prefetch(c):
        days_v, items_v, *_, isem, _, _ = bufs[c % 2]
        pltpu.async_copy(days_hbm.at[sl(c)], days_v, isem)
        pltpu.async_copy(items_hbm.at[sl(c)], items_v, isem)

    def stage(c):
        days_v, items_v, price_v, idx_v, _, mg_v, _, gsem, isem, msem, _ = (
            bufs[c % 2])
        pltpu.make_async_copy(days_hbm.at[sl(c)], days_v, isem).wait()
        pltpu.make_async_copy(items_hbm.at[sl(c)], items_v, isem).wait()
        if c >= 2:
            pltpu.make_async_copy(mg_v, om_hbm.at[sl(c)], msem).wait()

        def idx_loop(i, carry):
            u = i // gpr
            s = pl.ds((i % gpr) * _LANES, _LANES)
            it = items_v[u, s]
            dy = days_v[u, s]
            idx_v[pl.ds(i * _LANES, _LANES)] = (
                ((it >> 9) << 20) + ((dy >> 3) << 12)
                + (((it >> 7) & 3) << 10) + ((dy & 7) << 7) + (it & 127)
            )
            mg_v[u, s] = plsc.load_gather(mean_v, [it])
            return carry

        lax.fori_loop(0, _CHUNK // _LANES, idx_loop, 0)
        pltpu.async_copy(tbl_hbm.at[idx_v], price_v, gsem)
        pltpu.async_copy(mg_v, om_hbm.at[sl(c)], msem)

    def drain(c):
        _, _, price_v, idx_v, pg_v, mg_v, rg_v, gsem, _, _, osem = bufs[c % 2]
        pltpu.make_async_copy(tbl_hbm.at[idx_v], price_v, gsem).wait()
        if c >= 2:
            pltpu.make_async_copy(pg_v, op_hbm.at[sl(c)], osem).wait()
            pltpu.make_async_copy(rg_v, or_hbm.at[sl(c)], osem).wait()

        def div_loop(i, carry):
            u = i // gpr
            s = pl.ds((i % gpr) * _LANES, _LANES)
            p = price_v[pl.ds(i * _LANES, _LANES)]
            pg_v[u, s] = p
            rg_v[u, s] = p / mg_v[u, s]
            return carry

        lax.fori_loop(0, _CHUNK // _LANES, div_loop, 0)
        pltpu.async_copy(pg_v, op_hbm.at[sl(c)], osem)
        pltpu.async_copy(rg_v, or_hbm.at[sl(c)], osem)

    prefetch(0)
    prefetch(1)
    stage(0)
    stage(1)
    for c in range(n_chunks):
        if c + 2 < n_chunks:
            prefetch(c + 2)
        drain(c)
        if c + 2 < n_chunks:
            stage(c + 2)
    for c in (n_chunks - 2, n_chunks - 1):
        _, _, _, _, pg_v, mg_v, rg_v, _, _, msem, osem = bufs[c % 2]
        pltpu.make_async_copy(mg_v, om_hbm.at[sl(c)], msem).wait()
        pltpu.make_async_copy(pg_v, op_hbm.at[sl(c)], osem).wait()
        pltpu.make_async_copy(rg_v, or_hbm.at[sl(c)], osem).wait()


@functools.partial(jax.jit, static_argnames=("l", "b"))
def _sc_gather(tbl_flat, days_t, items_t, mean_pad, *, l, b):
    assert l % _CR == 0 and b == _CW * _NW
    n_chunks = l // _CR
    mesh = plsc.VectorSubcoreMesh(core_axis_name="c", subcore_axis_name="s")
    out = jax.ShapeDtypeStruct((l, b), jnp.float32)
    k = pl.kernel(
        functools.partial(_sc_body, n_chunks),
        out_type=(out, out, out),
        mesh=mesh,
        compiler_params=pltpu.CompilerParams(needs_layout_passes=False),
        scratch_types=[
            pltpu.VMEM((_ITEMS_PAD,), jnp.float32),
        ] + 2 * [
            pltpu.VMEM((_CR, _CW), jnp.int32),
            pltpu.VMEM((_CR, _CW), jnp.int32),
            pltpu.VMEM((_CHUNK,), jnp.float32),
            pltpu.VMEM((_CHUNK,), jnp.int32),
            pltpu.VMEM((_CR, _CW), jnp.float32),
            pltpu.VMEM((_CR, _CW), jnp.float32),
            pltpu.VMEM((_CR, _CW), jnp.float32),
            pltpu.SemaphoreType.DMA,
            pltpu.SemaphoreType.DMA,
            pltpu.SemaphoreType.DMA,
            pltpu.SemaphoreType.DMA,
        ],
    )
    return k(tbl_flat, days_t, items_t, mean_pad)


def kernel(prices, days_index, items_index):
    b, l = days_index.shape
    mean_pad, tbl_flat = _mean_and_flat(prices.T)
    gp, gm, gr = _sc_gather(
        tbl_flat,
        days_index.T.astype(jnp.int32),
        items_index.T.astype(jnp.int32),
        mean_pad,
        l=l, b=b,
    )
    return gp.T, gm.T, gr.T


# speedup vs baseline: 1.8559x; 1.0189x over previous
import functools

import jax
import jax.numpy as jnp
from jax import lax
from jax.experimental import pallas as pl
from jax.experimental.pallas import tpu as pltpu
from jax.experimental.pallas import tpu_sc as plsc

N_ITEMS = 30490
N_DAYS = 1969

_CB = 1024
_GRID = 30
_ITEMS_PAD = _CB * _GRID
_DPAD = 2048
_TBLK = _DPAD * _CB
_TSIZE = _GRID * _TBLK

_NW = 32
_LANES = 16
_CR = 8
_CW = 512
_CHUNK = _CR * _CW


def _tc_body(pt_ref, mean_ref, tbl_ref):
    x = pt_ref[...]
    mean_ref[...] = jnp.mean(x, axis=0)
    xp = jnp.concatenate(
        [x, jnp.zeros((_DPAD - N_DAYS, _CB), jnp.float32)], axis=0)
    y = xp.reshape(_DPAD // 8, 8, _CB // 128, 128).transpose(0, 2, 1, 3)
    tbl_ref[...] = y.reshape(_TBLK)


def _mean_and_flat(prices_t):
    return pl.pallas_call(
        _tc_body,
        grid=(_GRID,),
        in_specs=[pl.BlockSpec((N_DAYS, _CB), lambda i: (0, i))],
        out_specs=[
            pl.BlockSpec((_CB,), lambda i: (i,)),
            pl.BlockSpec((_TBLK,), lambda i: (i,)),
        ],
        out_shape=[
            jax.ShapeDtypeStruct((_ITEMS_PAD,), jnp.float32),
            jax.ShapeDtypeStruct((_TSIZE,), jnp.float32),
        ],
    )(prices_t)


def _sc_body(n_chunks, tbl_hbm, days_hbm, items_hbm, mean_hbm,
             op_hbm, om_hbm, or_hbm, mean_v,
             days_a, items_a, price_a, idx_a, pg_a, mg_a, rg_a,
             gsem_a, isem_a, msem_a, osem_a,
             days_b, items_b, price_b, idx_b, pg_b, mg_b, rg_b,
             gsem_b, isem_b, msem_b, osem_b):
    wid = lax.axis_index("s") * 2 + lax.axis_index("c")
    b0 = wid * _CW
    pltpu.sync_copy(mean_hbm, mean_v)
    bufs = (
        (days_a, items_a, price_a, idx_a, pg_a, mg_a, rg_a,
         gsem_a, isem_a, msem_a, osem_a),
        (days_b, items_b, price_b, idx_b, pg_b, mg_b, rg_b,
         gsem_b, isem_b, msem_b, osem_b),
    )
    gpr = _CW // _LANES

    def sl(c):
        return (pl.ds(c * _CR, _CR), pl.ds(b0, _CW))

    def prefetch(c):
        days_v, items_v, *_, isem, _, _ = bufs[c % 2]
        pltpu.async_copy(days_hbm.at[sl(c)], days_v, isem)
        pltpu.async_copy(items_hbm.at[sl(c)], items_v, isem)

    def stage(c):
        days_v, items_v, price_v, idx_v, _, mg_v, _, gsem, isem, msem, _ = (
            bufs[c % 2])
        pltpu.make_async_copy(days_hbm.at[sl(c)], days_v, isem).wait()
        pltpu.make_async_copy(items_hbm.at[sl(c)], items_v, isem).wait()
        if c >= 2:
            pltpu.make_async_copy(mg_v, om_hbm.at[sl(c)], msem).wait()

        def idx_loop(i, carry):
            u = i // gpr
            s = pl.ds((i % gpr) * _LANES, _LANES)
            it = items_v[u, s]
            dy = days_v[u, s]
            idx_v[pl.ds(i * _LANES, _LANES)] = (
                ((it >> 10) << 21) + ((dy >> 3) << 13)
                + (((it >> 7) & 7) << 10) + ((dy & 7) << 7) + (it & 127)
            )
            mg_v[u, s] = plsc.load_gather(mean_v, [it])
            return carry

        lax.fori_loop(0, _CHUNK // _LANES, idx_loop, 0)
        pltpu.async_copy(tbl_hbm.at[idx_v], price_v, gsem)
        pltpu.async_copy(mg_v, om_hbm.at[sl(c)], msem)

    def drain(c):
        _, _, price_v, idx_v, pg_v, mg_v, rg_v, gsem, _, _, osem = bufs[c % 2]
        pltpu.make_async_copy(tbl_hbm.at[idx_v], price_v, gsem).wait()
        if c >= 2:
            pltpu.make_async_copy(pg_v, op_hbm.at[sl(c)], osem).wait()
            pltpu.make_async_copy(rg_v, or_hbm.at[sl(c)], osem).wait()

        def div_loop(i, carry):
            u = i // gpr
            s = pl.ds((i % gpr) * _LANES, _LANES)
            p = price_v[pl.ds(i * _LANES, _LANES)]
            pg_v[u, s] = p
            rg_v[u, s] = p / mg_v[u, s]
            return carry

        lax.fori_loop(0, _CHUNK // _LANES, div_loop, 0)
        pltpu.async_copy(pg_v, op_hbm.at[sl(c)], osem)
        pltpu.async_copy(rg_v, or_hbm.at[sl(c)], osem)

    prefetch(0)
    prefetch(1)
    stage(0)
    stage(1)
    for c in range(n_chunks):
        if c + 2 < n_chunks:
            prefetch(c + 2)
        drain(c)
        if c + 2 < n_chunks:
            stage(c + 2)
    for c in (n_chunks - 2, n_chunks - 1):
        _, _, _, _, pg_v, mg_v, rg_v, _, _, msem, osem = bufs[c % 2]
        pltpu.make_async_copy(mg_v, om_hbm.at[sl(c)], msem).wait()
        pltpu.make_async_copy(pg_v, op_hbm.at[sl(c)], osem).wait()
        pltpu.make_async_copy(rg_v, or_hbm.at[sl(c)], osem).wait()


@functools.partial(jax.jit, static_argnames=("l", "b"))
def _sc_gather(tbl_flat, days_t, items_t, mean_pad, *, l, b):
    assert l % _CR == 0 and b == _CW * _NW
    n_chunks = l // _CR
    mesh = plsc.VectorSubcoreMesh(core_axis_name="c", subcore_axis_name="s")
    out = jax.ShapeDtypeStruct((l, b), jnp.float32)
    k = pl.kernel(
        functools.partial(_sc_body, n_chunks),
        out_type=(out, out, out),
        mesh=mesh,
        compiler_params=pltpu.CompilerParams(needs_layout_passes=False),
        scratch_types=[
            pltpu.VMEM((_ITEMS_PAD,), jnp.float32),
        ] + 2 * [
            pltpu.VMEM((_CR, _CW), jnp.int32),
            pltpu.VMEM((_CR, _CW), jnp.int32),
            pltpu.VMEM((_CHUNK,), jnp.float32),
            pltpu.VMEM((_CHUNK,), jnp.int32),
            pltpu.VMEM((_CR, _CW), jnp.float32),
            pltpu.VMEM((_CR, _CW), jnp.float32),
            pltpu.VMEM((_CR, _CW), jnp.float32),
            pltpu.SemaphoreType.DMA,
            pltpu.SemaphoreType.DMA,
            pltpu.SemaphoreType.DMA,
            pltpu.SemaphoreType.DMA,
        ],
    )
    return k(tbl_flat, days_t, items_t, mean_pad)


def kernel(prices, days_index, items_index):
    b, l = days_index.shape
    mean_pad, tbl_flat = _mean_and_flat(prices.T)
    gp, gm, gr = _sc_gather(
        tbl_flat,
        days_index.T.astype(jnp.int32),
        items_index.T.astype(jnp.int32),
        mean_pad,
        l=l, b=b,
    )
    return gp.T, gm.T, gr.T
